# baseline (speedup 1.0000x reference)
import jax
import jax.numpy as jnp
import numpy as np
from jax import lax
from jax.experimental import pallas as pl

_BLK = 299008

_KS0 = 0
_KS1 = 42
_KS2 = _KS0 ^ _KS1 ^ 0x1BD11BDA

_ROTS = ((13, 15, 26, 6), (17, 29, 16, 24))


def _rotl(x, r):
    return (x << jnp.int32(r)) | lax.shift_right_logical(x, jnp.int32(32 - r))


def _threefry_scale(x1):
    ks = (_KS0, _KS1, _KS2)
    x0 = None
    for i in range(5):
        for j, r in enumerate(_ROTS[i % 2]):
            x0 = x1 if x0 is None else x0 + x1
            x1 = x0 ^ _rotl(x1, r)
        a = ks[(i + 1) % 3]
        if a:
            x0 = x0 + jnp.int32(a)
        x1 = x1 + jnp.int32(ks[(i + 2) % 3] + i + 1)
    bits = x0 ^ x1
    return jnp.where(bits >= 0, jnp.float32(2.0), jnp.float32(0.0))


def _body(ramp_ref, v_ref, o_ref):
    o_ref[...] = v_ref[...] * _threefry_scale(ramp_ref[...])


@jax.jit
def _sparse_dropout(values):
    n = values.shape[0]
    grid = pl.cdiv(n, _BLK)
    ramp = np.arange(n, dtype=np.int32) + np.int32(_KS1)
    return pl.pallas_call(
        _body,
        grid=(grid,),
        in_specs=[
            pl.BlockSpec((_BLK,), lambda i: (i,)),
            pl.BlockSpec((_BLK,), lambda i: (i,)),
        ],
        out_specs=pl.BlockSpec((_BLK,), lambda i: (i,)),
        out_shape=jax.ShapeDtypeStruct((n,), jnp.float32),
    )(ramp, values)


def kernel(indices, values):
    del indices
    return _sparse_dropout(values)
